# baseline (speedup 1.0000x reference)
import functools

import jax
import jax.numpy as jnp
from jax import lax
from jax.experimental import pallas as pl
from jax.experimental.pallas import tpu as pltpu
from jax.experimental.pallas import tpu_sc as plsc

_VOCAB = 100000
_DIM = 64
_BATCH = 1024
_VB = 6144
_NV = (_VOCAB + _VB - 1) // _VB
_NEG = -1e30


_NROW = _DIM * _BATCH // 128
_RPW = _NROW // 32


def _sc_gather_t(table_flat, offs):
    info = plsc.get_sparse_core_info()
    mesh = plsc.VectorSubcoreMesh(core_axis_name="c", subcore_axis_name="s")

    @functools.partial(
        pl.kernel,
        mesh=mesh,
        out_type=jax.ShapeDtypeStruct((_NROW, 128), jnp.float32),
        scratch_types=[
            pltpu.VMEM((_RPW, 128), jnp.int32),
            pltpu.VMEM((_RPW, 128), jnp.float32),
            pltpu.SemaphoreType.DMA,
        ],
        compiler_params=pltpu.CompilerParams(use_tc_tiling_on_sc=False),
    )
    def gather_k(tbl_hbm, offs_hbm, out_hbm, offs_v, rows_v, sem):
        wid = lax.axis_index("s") * info.num_cores + lax.axis_index("c")
        base = wid * _RPW
        pltpu.sync_copy(offs_hbm.at[pl.ds(base, _RPW)], offs_v)
        copies = [
            pltpu.async_copy(tbl_hbm.at[offs_v.at[j]], rows_v.at[j], sem)
            for j in range(_RPW)
        ]
        for c in copies:
            c.wait()
        pltpu.sync_copy(rows_v, out_hbm.at[pl.ds(base, _RPW)])

    return gather_k(table_flat, offs)


_KE = _DIM + 4
_Q = 4
_BQ = _BATCH // _Q
_LOG2E = 1.4426950408889634
_LN2 = 0.6931471805599453


def _fused_body(e_ref, wt_ref, b_ref, o_ref, m_s, s_s, wt_s, ee_s, en_s, wn_s):
    p = pl.program_id(0)
    v = pl.program_id(1)

    @pl.when((p == 0) & (v == 0))
    def _prep_e():
        ef = e_ref[...]
        ee_s[pl.ds(0, _DIM), :] = ef.astype(jnp.bfloat16)
        ee_s[pl.ds(_DIM, 2), :] = jnp.ones((2, _BATCH), jnp.bfloat16)
        ee_s[pl.ds(_DIM + 2, 2), :] = jnp.zeros((2, _BATCH), jnp.bfloat16)
        esq = lax.dot_general(
            jnp.ones((1, _DIM), jnp.float32), ef * ef, (((1,), (0,)), ((), ())),
            preferred_element_type=jnp.float32)
        en_s[...] = jnp.sqrt(esq + 2.0)

    def _dot(wt_ext, lane0, width):
        return lax.dot_general(
            wt_ext, ee_s[:, pl.ds(lane0, width)], (((0,), (0,)), ((), ())),
            preferred_element_type=jnp.float32)

    def _stats(logits2, wnorm, ch, tail):
        sl = pl.ds(ch * _BQ, _BQ)

        @pl.when(v == 0)
        def _():
            m_s[:, sl] = jnp.zeros((1, _BQ), jnp.float32)
            s_s[:, sl] = jnp.zeros((1, _BQ), jnp.float32)

        if tail:
            col = v * _VB + lax.broadcasted_iota(jnp.int32, (1, _VB), 1)
            wnorm = jnp.where(col < _VOCAB, wnorm, 0.0)
        wbm = jnp.max(wnorm, axis=1, keepdims=True)
        m_old = m_s[:, sl]
        m_new = jnp.maximum(m_old, wbm * en_s[:, sl])
        eb = jnp.exp2(logits2 - m_new)
        if tail:
            row = v * _VB + lax.broadcasted_iota(jnp.int32, (_VB, 1), 0)
            eb = jnp.where(row < _VOCAB, eb, 0.0)
        ssum = lax.dot_general(
            jnp.ones((1, _VB), jnp.float32), eb, (((1,), (0,)), ((), ())),
            preferred_element_type=jnp.float32)
        s_s[:, sl] = s_s[:, sl] * jnp.exp2(m_old - m_new) + ssum
        m_s[:, sl] = m_new

        if tail:
            lse2 = m_new + jnp.log(s_s[:, sl]) * _LOG2E
            lse2_hi = lse2.astype(jnp.bfloat16)
            lse2_lo = (lse2 - lse2_hi.astype(jnp.float32)).astype(jnp.bfloat16)
            ee_s[pl.ds(_DIM + 2, 1), sl] = -lse2_hi
            ee_s[pl.ds(_DIM + 3, 1), sl] = -lse2_lo

    def _stats_both(logits2, wnorm, ch):
        @pl.when(v < _NV - 1)
        def _():
            _stats(logits2, wnorm, ch, tail=False)

        @pl.when(v == _NV - 1)
        def _():
            _stats(logits2, wnorm, ch, tail=True)

    @pl.when(p == 0)
    def _phase0():
        wts = wt_ref[...] * _LOG2E
        bb = b_ref[...] * _LOG2E
        b_hi = bb.astype(jnp.bfloat16)
        b_lo = (bb - b_hi.astype(jnp.float32)).astype(jnp.bfloat16)
        wt_ext = jnp.concatenate(
            [wts.astype(jnp.bfloat16), b_hi, b_lo,
             jnp.ones((2, _VB), jnp.bfloat16)], axis=0)
        wt_s[:, pl.ds(v * _VB, _VB)] = wt_ext
        wsq = lax.dot_general(
            jnp.ones((1, _DIM), jnp.float32), wts * wts,
            (((1,), (0,)), ((), ())), preferred_element_type=jnp.float32)
        wnorm = jnp.sqrt(wsq + bb * bb)
        wn_s[:, pl.ds(v * _VB, _VB)] = wnorm
        _stats_both(_dot(wt_ext, 0, _BQ), wnorm, 0)

    for _ph in range(1, _Q):
        @pl.when(p == _ph)
        def _phase_mid(ph=_ph):
            wt_ext = wt_s[:, pl.ds(v * _VB, _VB)]
            lw = _dot(wt_ext, (ph - 1) * _BQ, 2 * _BQ)
            o_ref[...] = lw[:, 0:_BQ] * _LN2
            _stats_both(lw[:, _BQ:2 * _BQ],
                        wn_s[:, pl.ds(v * _VB, _VB)], ph)

    @pl.when(p == _Q)
    def _phase_last():
        wt_ext = wt_s[:, pl.ds(v * _VB, _VB)]
        o_ref[...] = _dot(wt_ext, (_Q - 1) * _BQ, _BQ) * _LN2


def kernel(target_word, emb_table, W, b):
    idx = target_word.astype(jnp.int32)
    table_flat = emb_table.T.reshape(_VOCAB * _DIM)
    offs = (jnp.arange(_DIM, dtype=jnp.int32) * _VOCAB)[:, None] + idx[None, :]
    e_t = _sc_gather_t(table_flat, offs.reshape(_NROW, 128))
    e_t = e_t.reshape(_DIM, _BATCH)
    Wt = W.T
    b2 = b.reshape(1, _VOCAB)

    out_t = pl.pallas_call(
        _fused_body,
        grid=(_Q + 1, _NV),
        in_specs=[
            pl.BlockSpec((_DIM, _BATCH), lambda p, v: (0, 0)),
            pl.BlockSpec((_DIM, _VB),
                         lambda p, v: (0, jnp.where(p == 0, v, _NV - 1))),
            pl.BlockSpec((1, _VB),
                         lambda p, v: (0, jnp.where(p == 0, v, _NV - 1))),
        ],
        out_specs=pl.BlockSpec(
            (_VB, _BQ),
            lambda p, v: (jnp.where(p == 0, 0, v), jnp.maximum(p - 1, 0))),
        out_shape=jax.ShapeDtypeStruct((_VOCAB, _BATCH), jnp.float32),
        scratch_shapes=[
            pltpu.VMEM((1, _BATCH), jnp.float32),
            pltpu.VMEM((1, _BATCH), jnp.float32),
            pltpu.VMEM((_KE, _NV * _VB), jnp.bfloat16),
            pltpu.VMEM((_KE, _BATCH), jnp.bfloat16),
            pltpu.VMEM((1, _BATCH), jnp.float32),
            pltpu.VMEM((1, _NV * _VB), jnp.float32),
        ],
        compiler_params=pltpu.CompilerParams(
            dimension_semantics=("arbitrary", "arbitrary")),
    )(e_t, Wt, b2)
    return out_t.T


# speedup vs baseline: 1.0079x; 1.0079x over previous
import functools

import jax
import jax.numpy as jnp
from jax import lax
from jax.experimental import pallas as pl
from jax.experimental.pallas import tpu as pltpu
from jax.experimental.pallas import tpu_sc as plsc

_VOCAB = 100000
_DIM = 64
_BATCH = 1024
_VB = 6144
_NV = (_VOCAB + _VB - 1) // _VB
_NEG = -1e30


_NROW = _DIM * _BATCH // 128
_RPW = _NROW // 32


def _sc_gather_t(table_flat, offs):
    info = plsc.get_sparse_core_info()
    mesh = plsc.VectorSubcoreMesh(core_axis_name="c", subcore_axis_name="s")

    @functools.partial(
        pl.kernel,
        mesh=mesh,
        out_type=jax.ShapeDtypeStruct((_NROW, 128), jnp.float32),
        scratch_types=[
            pltpu.VMEM((_RPW, 128), jnp.int32),
            pltpu.VMEM((_RPW, 128), jnp.float32),
            pltpu.SemaphoreType.DMA,
        ],
        compiler_params=pltpu.CompilerParams(use_tc_tiling_on_sc=False),
    )
    def gather_k(tbl_hbm, offs_hbm, out_hbm, offs_v, rows_v, sem):
        wid = lax.axis_index("s") * info.num_cores + lax.axis_index("c")
        base = wid * _RPW
        pltpu.sync_copy(offs_hbm.at[pl.ds(base, _RPW)], offs_v)
        copies = [
            pltpu.async_copy(tbl_hbm.at[offs_v.at[j]], rows_v.at[j], sem)
            for j in range(_RPW)
        ]
        for c in copies:
            c.wait()
        pltpu.sync_copy(rows_v, out_hbm.at[pl.ds(base, _RPW)])

    return gather_k(table_flat, offs)


_KE = _DIM + 4
_Q = 2
_BQ = _BATCH // _Q
_LOG2E = 1.4426950408889634
_LN2 = 0.6931471805599453


def _fused_body(e_ref, wt_ref, b_ref, o_ref, m_s, s_s, wt_s, ee_s, en_s, wn_s):
    p = pl.program_id(0)
    v = pl.program_id(1)

    @pl.when((p == 0) & (v == 0))
    def _prep_e():
        ef = e_ref[...]
        ee_s[pl.ds(0, _DIM), :] = ef.astype(jnp.bfloat16)
        ee_s[pl.ds(_DIM, 2), :] = jnp.ones((2, _BATCH), jnp.bfloat16)
        ee_s[pl.ds(_DIM + 2, 2), :] = jnp.zeros((2, _BATCH), jnp.bfloat16)
        esq = lax.dot_general(
            jnp.ones((1, _DIM), jnp.float32), ef * ef, (((1,), (0,)), ((), ())),
            preferred_element_type=jnp.float32)
        en_s[...] = jnp.sqrt(esq + 2.0)

    def _dot(wt_ext, lane0, width):
        return lax.dot_general(
            wt_ext, ee_s[:, pl.ds(lane0, width)], (((0,), (0,)), ((), ())),
            preferred_element_type=jnp.float32)

    def _stats(logits2, wnorm, ch, tail):
        sl = pl.ds(ch * _BQ, _BQ)

        @pl.when(v == 0)
        def _():
            m_s[:, sl] = jnp.zeros((1, _BQ), jnp.float32)
            s_s[:, sl] = jnp.zeros((1, _BQ), jnp.float32)

        if tail:
            col = v * _VB + lax.broadcasted_iota(jnp.int32, (1, _VB), 1)
            wnorm = jnp.where(col < _VOCAB, wnorm, 0.0)
        wbm = jnp.max(wnorm, axis=1, keepdims=True)
        m_old = m_s[:, sl]
        m_new = jnp.maximum(m_old, wbm * en_s[:, sl])
        eb = jnp.exp2(logits2 - m_new)
        if tail:
            row = v * _VB + lax.broadcasted_iota(jnp.int32, (_VB, 1), 0)
            eb = jnp.where(row < _VOCAB, eb, 0.0)
        ssum = lax.dot_general(
            jnp.ones((1, _VB), jnp.float32), eb, (((1,), (0,)), ((), ())),
            preferred_element_type=jnp.float32)
        s_s[:, sl] = s_s[:, sl] * jnp.exp2(m_old - m_new) + ssum
        m_s[:, sl] = m_new

        if tail:
            lse2 = m_new + jnp.log(s_s[:, sl]) * _LOG2E
            lse2_hi = lse2.astype(jnp.bfloat16)
            lse2_lo = (lse2 - lse2_hi.astype(jnp.float32)).astype(jnp.bfloat16)
            ee_s[pl.ds(_DIM + 2, 1), sl] = -lse2_hi
            ee_s[pl.ds(_DIM + 3, 1), sl] = -lse2_lo

    def _stats_both(logits2, wnorm, ch):
        @pl.when(v < _NV - 1)
        def _():
            _stats(logits2, wnorm, ch, tail=False)

        @pl.when(v == _NV - 1)
        def _():
            _stats(logits2, wnorm, ch, tail=True)

    @pl.when(p == 0)
    def _phase0():
        wts = wt_ref[...] * _LOG2E
        bb = b_ref[...] * _LOG2E
        b_hi = bb.astype(jnp.bfloat16)
        b_lo = (bb - b_hi.astype(jnp.float32)).astype(jnp.bfloat16)
        wt_ext = jnp.concatenate(
            [wts.astype(jnp.bfloat16), b_hi, b_lo,
             jnp.ones((2, _VB), jnp.bfloat16)], axis=0)
        wt_s[:, pl.ds(v * _VB, _VB)] = wt_ext
        wsq = lax.dot_general(
            jnp.ones((1, _DIM), jnp.float32), wts * wts,
            (((1,), (0,)), ((), ())), preferred_element_type=jnp.float32)
        wnorm = jnp.sqrt(wsq + bb * bb)
        wn_s[:, pl.ds(v * _VB, _VB)] = wnorm
        _stats_both(_dot(wt_ext, 0, _BQ), wnorm, 0)

    for _ph in range(1, _Q):
        @pl.when(p == _ph)
        def _phase_mid(ph=_ph):
            wt_ext = wt_s[:, pl.ds(v * _VB, _VB)]
            lw = _dot(wt_ext, (ph - 1) * _BQ, 2 * _BQ)
            o_ref[...] = lw[:, 0:_BQ] * _LN2
            _stats_both(lw[:, _BQ:2 * _BQ],
                        wn_s[:, pl.ds(v * _VB, _VB)], ph)

    @pl.when(p == _Q)
    def _phase_last():
        wt_ext = wt_s[:, pl.ds(v * _VB, _VB)]
        o_ref[...] = _dot(wt_ext, (_Q - 1) * _BQ, _BQ) * _LN2


def kernel(target_word, emb_table, W, b):
    idx = target_word.astype(jnp.int32)
    table_flat = emb_table.T.reshape(_VOCAB * _DIM)
    offs = (jnp.arange(_DIM, dtype=jnp.int32) * _VOCAB)[:, None] + idx[None, :]
    e_t = _sc_gather_t(table_flat, offs.reshape(_NROW, 128))
    e_t = e_t.reshape(_DIM, _BATCH)
    Wt = W.T
    b2 = b.reshape(1, _VOCAB)

    out_t = pl.pallas_call(
        _fused_body,
        grid=(_Q + 1, _NV),
        in_specs=[
            pl.BlockSpec((_DIM, _BATCH), lambda p, v: (0, 0)),
            pl.BlockSpec((_DIM, _VB),
                         lambda p, v: (0, jnp.where(p == 0, v, _NV - 1))),
            pl.BlockSpec((1, _VB),
                         lambda p, v: (0, jnp.where(p == 0, v, _NV - 1))),
        ],
        out_specs=pl.BlockSpec(
            (_VB, _BQ),
            lambda p, v: (jnp.where(p == 0, 0, v), jnp.maximum(p - 1, 0))),
        out_shape=jax.ShapeDtypeStruct((_VOCAB, _BATCH), jnp.float32),
        scratch_shapes=[
            pltpu.VMEM((1, _BATCH), jnp.float32),
            pltpu.VMEM((1, _BATCH), jnp.float32),
            pltpu.VMEM((_KE, _NV * _VB), jnp.bfloat16),
            pltpu.VMEM((_KE, _BATCH), jnp.bfloat16),
            pltpu.VMEM((1, _BATCH), jnp.float32),
            pltpu.VMEM((1, _NV * _VB), jnp.float32),
        ],
        compiler_params=pltpu.CompilerParams(
            dimension_semantics=("arbitrary", "arbitrary")),
    )(e_t, Wt, b2)
    return out_t.T
